# baseline (speedup 1.0000x reference)
import numpy as np
import jax
import jax.numpy as jnp
from jax import lax
from jax.experimental import pallas as pl
from jax.experimental.pallas import tpu as pltpu
from jax.experimental.pallas import tpu_sc as plsc

N = 16384
K = 512
D = 32
CH = 512
R = N // CH
NW = 32
PW = N // NW
DP = 48
L = 16
_Q = float(np.arctanh(0.5) ** 2 + 1.0)


def _sc_body(coords_hbm, truth_hbm, sums_hbm, rows_v, tru_v, acc_v):
    cid = lax.axis_index("c")
    sid = lax.axis_index("s")
    w = cid * 16 + sid
    pltpu.sync_copy(coords_hbm.at[pl.ds(w * PW * DP, PW * DP)], rows_v)
    pltpu.sync_copy(truth_hbm.at[pl.ds(w * PW, PW)], tru_v)

    zer = jnp.zeros((L,), jnp.float32)

    def zero_row(i, _):
        acc_v[pl.ds(i * L, L)] = zer
        return 0

    lax.fori_loop(0, K * DP // L, zero_row, 0)

    def accum(b, _):
        tvec = tru_v[pl.ds(b * L, L)]
        for p in range(L):
            t = tvec[p]
            j = b * L + p
            acc_v[pl.ds(t * DP, L)] += rows_v[pl.ds(j * DP, L)]
            acc_v[pl.ds(t * DP + L, L)] += rows_v[pl.ds(j * DP + L, L)]
            acc_v[pl.ds(t * DP + 2 * L, L)] += rows_v[pl.ds(j * DP + 2 * L, L)]
        return 0

    lax.fori_loop(0, PW // L, accum, 0)
    pltpu.sync_copy(acc_v, sums_hbm.at[w])


def _sc_segstats(coords_pad_flat, truth_flat):
    run = pl.kernel(
        _sc_body,
        out_type=jax.ShapeDtypeStruct((NW, K * DP), jnp.float32),
        mesh=plsc.VectorSubcoreMesh(core_axis_name="c", subcore_axis_name="s"),
        scratch_types=[
            pltpu.VMEM((PW * DP,), jnp.float32),
            pltpu.VMEM((PW,), jnp.int32),
            pltpu.VMEM((K * DP,), jnp.float32),
        ],
    )
    return run(coords_pad_flat, truth_flat)


def _dense_body(coords_ref, truth_ref, parts_ref, out_ref):
    f32 = jnp.float32

    def fold(w, acc):
        return acc + parts_ref[w]

    part = lax.fori_loop(0, NW, fold, jnp.zeros((K, DP), f32))
    cc_sum = part[:, :D]
    counts = jnp.sum(part[:, D:], axis=1, keepdims=True) * (1.0 / L)
    denom = jnp.maximum(_Q * counts, 1e-6)
    cc = (_Q * cc_sum) / denom
    ccn = jnp.sum(cc * cc, axis=1, keepdims=True)
    nonempty = (counts > 0).astype(f32)
    ones_row = jnp.ones((1, D), f32)

    def pb(r, tot):
        crd = coords_ref[pl.ds(r * CH, CH), :]
        trow = truth_ref[pl.ds(r, 1), :]
        own = lax.broadcasted_iota(jnp.int32, (K, CH), 0) == trow
        rn = lax.dot_general(
            ones_row, crd * crd, (((1,), (1,)), ((), ())),
            preferred_element_type=f32)
        dotm = lax.dot_general(
            cc, crd, (((1,), (1,)), ((), ())), preferred_element_type=f32)
        d2 = jnp.maximum(ccn + rn - 2.0 * dotm, 0.0)
        dist = jnp.sqrt(d2 + 1e-6)
        repm = jnp.maximum(0.0, 1.0 - dist)
        vals = jnp.where(own, d2, repm) * nonempty
        return tot + jnp.sum(vals)

    total = lax.fori_loop(0, R, pb, jnp.zeros((), f32))
    out_ref[0, 0] = total * (_Q * _Q / N)


def _dense(coords, truth_rows, parts, interpret=False):
    return pl.pallas_call(
        _dense_body,
        out_shape=jax.ShapeDtypeStruct((1, 1), jnp.float32),
        in_specs=[pl.BlockSpec(memory_space=pltpu.VMEM)] * 3,
        out_specs=pl.BlockSpec(memory_space=pltpu.SMEM),
        interpret=interpret,
    )(coords, truth_rows, parts)


def kernel(x, coords, truth, row_splits):
    truth_flat = truth.reshape(-1).astype(jnp.int32)
    coords_pad = jnp.concatenate(
        [coords, jnp.ones((N, L), jnp.float32)], axis=1)
    parts = _sc_segstats(coords_pad.reshape(-1), truth_flat)
    parts = parts.reshape(NW, K, DP)
    truth_rows = truth_flat.reshape(R, CH)
    out = _dense(coords, truth_rows, parts)
    return out[0, 0]


# speedup vs baseline: 1.2878x; 1.2878x over previous
import numpy as np
import jax
import jax.numpy as jnp
from jax import lax
from jax.experimental import pallas as pl
from jax.experimental.pallas import tpu as pltpu
from jax.experimental.pallas import tpu_sc as plsc

N = 16384
K = 512
D = 32
CH = 512
R = N // CH
NW = 32
PW = N // NW
HW = PW // 2
DP = 48
L = 16
_Q = float(np.arctanh(0.5) ** 2 + 1.0)


def _sc_body(coords_hbm, truth_hbm, sums_hbm, rows_v, tru_v, acc_v):
    cid = lax.axis_index("c")
    sid = lax.axis_index("s")
    w = cid * 16 + sid
    pltpu.sync_copy(truth_hbm.at[pl.ds(w, 1)], tru_v)

    zer = jnp.zeros((L,), jnp.float32)
    one = jnp.ones((L,), jnp.float32)

    def zero_rows(i, _):
        for u in range(8):
            r = i * 8 + u
            acc_v[r, pl.ds(0, L)] = zer
            acc_v[r, pl.ds(L, L)] = zer
            acc_v[r, pl.ds(2 * L, L)] = zer
        return 0

    lax.fori_loop(0, K // 8, zero_rows, 0)

    for half in range(2):
        base = w * PW + half * HW
        pltpu.sync_copy(coords_hbm.at[pl.ds(base, HW)], rows_v)

        def accum(b, _):
            tvec = tru_v[0, pl.ds(half * HW + b * L, L)]
            for p in range(L):
                t = tvec[p]
                j = b * L + p
                acc_v[t, pl.ds(0, L)] += rows_v[j, pl.ds(0, L)]
                acc_v[t, pl.ds(L, L)] += rows_v[j, pl.ds(L, L)]
                acc_v[t, pl.ds(2 * L, L)] += one
            return 0

        lax.fori_loop(0, HW // L, accum, 0)

    pltpu.sync_copy(acc_v, sums_hbm.at[w])


def _sc_segstats(coords, truth_rows):
    run = pl.kernel(
        _sc_body,
        out_type=jax.ShapeDtypeStruct((NW, K, DP), jnp.float32),
        mesh=plsc.VectorSubcoreMesh(core_axis_name="c", subcore_axis_name="s"),
        scratch_types=[
            pltpu.VMEM((HW, D), jnp.float32),
            pltpu.VMEM((1, PW), jnp.int32),
            pltpu.VMEM((K, DP), jnp.float32),
        ],
    )
    return run(coords, truth_rows)


def _dense_body(coords_ref, truth_ref, parts_ref, out_ref):
    f32 = jnp.float32

    def fold(w, acc):
        return acc + parts_ref[w]

    part = lax.fori_loop(0, NW, fold, jnp.zeros((K, DP), f32))
    cc_sum = part[:, :D]
    counts = jnp.sum(part[:, D:], axis=1, keepdims=True) * (1.0 / L)
    denom = jnp.maximum(_Q * counts, 1e-6)
    cc = (_Q * cc_sum) / denom
    ccn = jnp.sum(cc * cc, axis=1, keepdims=True)
    nonempty = (counts > 0).astype(f32)
    ones_row = jnp.ones((1, D), f32)

    def pb(r, tot):
        crd = coords_ref[pl.ds(r * CH, CH), :]
        trow = truth_ref[pl.ds(r, 1), :]
        own = lax.broadcasted_iota(jnp.int32, (K, CH), 0) == trow
        rn = lax.dot_general(
            ones_row, crd * crd, (((1,), (1,)), ((), ())),
            preferred_element_type=f32)
        dotm = lax.dot_general(
            cc, crd, (((1,), (1,)), ((), ())), preferred_element_type=f32)
        d2 = jnp.maximum(ccn + rn - 2.0 * dotm, 0.0)
        dist = jnp.sqrt(d2 + 1e-6)
        repm = jnp.maximum(0.0, 1.0 - dist)
        vals = jnp.where(own, d2, repm) * nonempty
        return tot + jnp.sum(vals)

    total = lax.fori_loop(0, R, pb, jnp.zeros((), f32))
    out_ref[0, 0] = total * (_Q * _Q / N)


def _dense(coords, truth_rows, parts, interpret=False):
    return pl.pallas_call(
        _dense_body,
        out_shape=jax.ShapeDtypeStruct((1, 1), jnp.float32),
        in_specs=[pl.BlockSpec(memory_space=pltpu.VMEM)] * 3,
        out_specs=pl.BlockSpec(memory_space=pltpu.SMEM),
        interpret=interpret,
    )(coords, truth_rows, parts)


def kernel(x, coords, truth, row_splits):
    truth_rows = truth.reshape(R, CH).astype(jnp.int32)
    parts = _sc_segstats(coords, truth_rows)
    out = _dense(coords, truth_rows, parts)
    return out[0, 0]


# speedup vs baseline: 1.3382x; 1.0391x over previous
import numpy as np
import jax
import jax.numpy as jnp
from jax import lax
from jax.experimental import pallas as pl
from jax.experimental.pallas import tpu as pltpu
from jax.experimental.pallas import tpu_sc as plsc

N = 16384
K = 512
D = 32
CH = 512
R = N // CH
NW = 32
PW = N // NW
HW = PW // 2
DP = 48
L = 16
_Q = float(np.arctanh(0.5) ** 2 + 1.0)


def _sc_body(coords_hbm, truth_hbm, sums_hbm, rows_v, tru_v, acc_v):
    cid = lax.axis_index("c")
    sid = lax.axis_index("s")
    w = cid * 16 + sid
    pltpu.sync_copy(truth_hbm.at[pl.ds(w, 1)], tru_v)

    zer = jnp.zeros((L,), jnp.float32)
    one = jnp.ones((L,), jnp.float32)

    def zero_rows(i, _):
        for u in range(8):
            r = i * 8 + u
            acc_v[r, pl.ds(0, L)] = zer
            acc_v[r, pl.ds(L, L)] = zer
            acc_v[r, pl.ds(2 * L, L)] = zer
        return 0

    lax.fori_loop(0, K // 8, zero_rows, 0)

    for half in range(2):
        base = w * PW + half * HW
        pltpu.sync_copy(coords_hbm.at[pl.ds(base, HW)], rows_v)

        def accum(b, _):
            tvec = tru_v[0, pl.ds(half * HW + b * L, L)]
            for p in range(L):
                t = tvec[p]
                j = b * L + p
                acc_v[t, pl.ds(0, L)] += rows_v[j, pl.ds(0, L)]
                acc_v[t, pl.ds(L, L)] += rows_v[j, pl.ds(L, L)]
                acc_v[t, pl.ds(2 * L, L)] += one
            return 0

        lax.fori_loop(0, HW // L, accum, 0)

    pltpu.sync_copy(acc_v, sums_hbm.at[w])


def _sc_segstats(coords, truth_rows):
    run = pl.kernel(
        _sc_body,
        out_type=jax.ShapeDtypeStruct((NW, K, DP), jnp.float32),
        mesh=plsc.VectorSubcoreMesh(core_axis_name="c", subcore_axis_name="s"),
        scratch_types=[
            pltpu.VMEM((HW, D), jnp.float32),
            pltpu.VMEM((1, PW), jnp.int32),
            pltpu.VMEM((K, DP), jnp.float32),
        ],
        compiler_params=pltpu.CompilerParams(use_tc_tiling_on_sc=True),
    )
    return run(coords, truth_rows)


def _dense_body(coords_ref, truth_ref, parts_ref, out_ref):
    f32 = jnp.float32

    def fold(w, acc):
        return acc + parts_ref[w]

    part = lax.fori_loop(0, NW, fold, jnp.zeros((K, DP), f32))
    cc_sum = part[:, :D]
    counts = jnp.sum(part[:, D:], axis=1, keepdims=True) * (1.0 / L)
    denom = jnp.maximum(_Q * counts, 1e-6)
    cc = (_Q * cc_sum) / denom
    ccn1 = jnp.sum(cc * cc, axis=1, keepdims=True) + 1e-6
    nonempty = (counts > 0).astype(f32)
    ones_row = jnp.ones((1, D), f32)
    iota2d = lax.broadcasted_iota(jnp.int32, (K, CH), 0)

    def pb(r, tot):
        crd = coords_ref[pl.ds(r * CH, CH), :]
        trow = truth_ref[pl.ds(r, 1), :]
        own = iota2d == trow
        rn = lax.dot_general(
            ones_row, crd * crd, (((1,), (1,)), ((), ())),
            preferred_element_type=f32)
        dotm = lax.dot_general(
            cc, crd, (((1,), (1,)), ((), ())), preferred_element_type=f32)
        d2c = jnp.maximum(ccn1 + rn - 2.0 * dotm, 0.0)
        dist = jnp.sqrt(d2c)
        repm = jnp.maximum(0.0, 1.0 - dist)
        vals = jnp.where(own, d2c, repm)
        return tot + lax.dot_general(
            nonempty, vals, (((0,), (0,)), ((), ())),
            preferred_element_type=f32)

    tot_row = lax.fori_loop(0, R, pb, jnp.zeros((1, CH), f32))
    out_ref[0, 0] = jnp.sum(tot_row) * (_Q * _Q / N)


def _dense(coords, truth_rows, parts, interpret=False):
    return pl.pallas_call(
        _dense_body,
        out_shape=jax.ShapeDtypeStruct((1, 1), jnp.float32),
        in_specs=[pl.BlockSpec(memory_space=pltpu.VMEM)] * 3,
        out_specs=pl.BlockSpec(memory_space=pltpu.SMEM),
        interpret=interpret,
    )(coords, truth_rows, parts)


def kernel(x, coords, truth, row_splits):
    truth_rows = truth.reshape(R, CH).astype(jnp.int32)
    parts = _sc_segstats(coords, truth_rows)
    out = _dense(coords, truth_rows, parts)
    return out[0, 0]


# speedup vs baseline: 1.3578x; 1.0146x over previous
import numpy as np
import jax
import jax.numpy as jnp
from jax import lax
from jax.experimental import pallas as pl
from jax.experimental.pallas import tpu as pltpu
from jax.experimental.pallas import tpu_sc as plsc

N = 16384
K = 512
D = 32
CH = 512
R = N // CH
NW = 32
PW = N // NW
HW = PW // 2
DP = 48
L = 16
_Q = float(np.arctanh(0.5) ** 2 + 1.0)


def _sc_body(coords_hbm, truth_hbm, sums_hbm, rows_v, tru_v, acc_v):
    cid = lax.axis_index("c")
    sid = lax.axis_index("s")
    w = cid * 16 + sid
    pltpu.sync_copy(truth_hbm.at[pl.ds(w, 1)], tru_v)

    zer = jnp.zeros((L,), jnp.float32)
    one = jnp.ones((L,), jnp.float32)

    def zero_rows(i, _):
        for u in range(8):
            r = i * 8 + u
            acc_v[r, pl.ds(0, L)] = zer
            acc_v[r, pl.ds(L, L)] = zer
            acc_v[r, pl.ds(2 * L, L)] = zer
        return 0

    lax.fori_loop(0, K // 8, zero_rows, 0)

    for half in range(2):
        base = w * PW + half * HW
        pltpu.sync_copy(coords_hbm.at[pl.ds(base, HW)], rows_v)

        def accum(b, _):
            tvec = tru_v[0, pl.ds(half * HW + b * L, L)]
            for p in range(L):
                t = tvec[p]
                j = b * L + p
                acc_v[t, pl.ds(0, L)] += rows_v[j, pl.ds(0, L)]
                acc_v[t, pl.ds(L, L)] += rows_v[j, pl.ds(L, L)]
                acc_v[t, pl.ds(2 * L, L)] += one
            return 0

        lax.fori_loop(0, HW // L, accum, 0)

    pltpu.sync_copy(acc_v, sums_hbm.at[w])


def _sc_segstats(coords, truth_rows):
    run = pl.kernel(
        _sc_body,
        out_type=jax.ShapeDtypeStruct((NW, K, DP), jnp.float32),
        mesh=plsc.VectorSubcoreMesh(core_axis_name="c", subcore_axis_name="s"),
        scratch_types=[
            pltpu.VMEM((HW, D), jnp.float32),
            pltpu.VMEM((1, PW), jnp.int32),
            pltpu.VMEM((K, DP), jnp.float32),
        ],
        compiler_params=pltpu.CompilerParams(use_tc_tiling_on_sc=True),
    )
    return run(coords, truth_rows)


def _dense_body(coords_ref, truth_ref, parts_ref, out_ref):
    f32 = jnp.float32

    def fold(w, acc):
        return acc + parts_ref[w]

    part = lax.fori_loop(0, NW, fold, jnp.zeros((K, DP), f32))
    cc_sum = part[:, :D]
    counts = jnp.sum(part[:, D:], axis=1, keepdims=True) * (1.0 / L)
    denom = jnp.maximum(_Q * counts, 1e-6)
    cc = (_Q * cc_sum) / denom
    ccn1 = jnp.sum(cc * cc, axis=1, keepdims=True) + 1e-6
    nonempty = (counts > 0).astype(f32)
    cc_aug = jnp.concatenate(
        [-2.0 * cc, ccn1, jnp.ones((K, 1), f32)], axis=1)
    ones_col = jnp.ones((CH, 1), f32)
    iota2d = lax.broadcasted_iota(jnp.int32, (K, CH), 0)

    def pb(r, tot):
        crd = coords_ref[pl.ds(r * CH, CH), :]
        trow = truth_ref[pl.ds(r, 1), :]
        own = iota2d == trow
        rn = jnp.sum(crd * crd, axis=1, keepdims=True)
        crd_aug = jnp.concatenate([crd, ones_col, rn], axis=1)
        d2c = jnp.maximum(lax.dot_general(
            cc_aug, crd_aug, (((1,), (1,)), ((), ())),
            preferred_element_type=f32), 0.0)
        dist = jnp.sqrt(d2c)
        repm = jnp.maximum(0.0, 1.0 - dist)
        vals = jnp.where(own, d2c, repm)
        return tot + lax.dot_general(
            nonempty, vals, (((0,), (0,)), ((), ())),
            preferred_element_type=f32)

    tot_row = lax.fori_loop(0, R, pb, jnp.zeros((1, CH), f32))
    out_ref[0, 0] = jnp.sum(tot_row) * (_Q * _Q / N)


def _dense(coords, truth_rows, parts, interpret=False):
    return pl.pallas_call(
        _dense_body,
        out_shape=jax.ShapeDtypeStruct((1, 1), jnp.float32),
        in_specs=[pl.BlockSpec(memory_space=pltpu.VMEM)] * 3,
        out_specs=pl.BlockSpec(memory_space=pltpu.SMEM),
        interpret=interpret,
    )(coords, truth_rows, parts)


def kernel(x, coords, truth, row_splits):
    truth_rows = truth.reshape(R, CH).astype(jnp.int32)
    parts = _sc_segstats(coords, truth_rows)
    out = _dense(coords, truth_rows, parts)
    return out[0, 0]


# speedup vs baseline: 1.4597x; 1.0750x over previous
import numpy as np
import jax
import jax.numpy as jnp
from jax import lax
from jax.experimental import pallas as pl
from jax.experimental.pallas import tpu as pltpu
from jax.experimental.pallas import tpu_sc as plsc

N = 16384
K = 512
D = 32
CH = 512
R = N // CH
NW = 32
PW = N // NW
HW = PW // 2
DP = 48
L = 16
_Q = float(np.arctanh(0.5) ** 2 + 1.0)


def _sc_body(coords_hbm, truth_hbm, sums_hbm, rows_v, tru_v, acc_v):
    cid = lax.axis_index("c")
    sid = lax.axis_index("s")
    w = cid * 16 + sid
    pltpu.sync_copy(truth_hbm.at[pl.ds(w, 1)], tru_v)

    zer = jnp.zeros((L,), jnp.float32)
    one = jnp.ones((L,), jnp.float32)

    def zero_rows(i, _):
        for u in range(8):
            r = i * 8 + u
            acc_v[r, pl.ds(0, L)] = zer
            acc_v[r, pl.ds(L, L)] = zer
            acc_v[r, pl.ds(2 * L, L)] = zer
        return 0

    lax.fori_loop(0, K // 8, zero_rows, 0)

    for half in range(2):
        base = w * PW + half * HW
        pltpu.sync_copy(coords_hbm.at[pl.ds(base, HW)], rows_v)

        def accum(b, _):
            tvec = tru_v[0, pl.ds(half * HW + b * L, L)]
            for p in range(L):
                t = tvec[p]
                j = b * L + p
                acc_v[t, pl.ds(0, L)] += rows_v[j, pl.ds(0, L)]
                acc_v[t, pl.ds(L, L)] += rows_v[j, pl.ds(L, L)]
                acc_v[t, pl.ds(2 * L, L)] += one
            return 0

        lax.fori_loop(0, HW // L, accum, 0)

    pltpu.sync_copy(acc_v, sums_hbm.at[w])


def _sc_segstats(coords, truth_rows):
    run = pl.kernel(
        _sc_body,
        out_type=jax.ShapeDtypeStruct((NW, K, DP), jnp.float32),
        mesh=plsc.VectorSubcoreMesh(core_axis_name="c", subcore_axis_name="s"),
        scratch_types=[
            pltpu.VMEM((HW, D), jnp.float32),
            pltpu.VMEM((1, PW), jnp.int32),
            pltpu.VMEM((K, DP), jnp.float32),
        ],
        compiler_params=pltpu.CompilerParams(use_tc_tiling_on_sc=True),
    )
    return run(coords, truth_rows)


def _dense_body(coords_ref, truth_ref, parts_ref, out_ref):
    f32 = jnp.float32

    def fold(w, acc):
        return acc + parts_ref[w]

    part = lax.fori_loop(0, NW, fold, jnp.zeros((K, DP), f32))
    cc_sum = part[:, :D]
    counts = jnp.sum(part[:, D:], axis=1, keepdims=True) * (1.0 / L)
    denom = jnp.maximum(_Q * counts, 1e-6)
    cc = (_Q * cc_sum) / denom
    ccn1 = jnp.sum(cc * cc, axis=1, keepdims=True) + 1e-6
    nonempty = (counts > 0).astype(f32)
    cc_aug = jnp.concatenate(
        [-2.0 * cc, ccn1, jnp.ones((K, 1), f32)], axis=1)
    ones_col = jnp.ones((CH, 1), f32)
    iota2d = lax.broadcasted_iota(jnp.int32, (K, CH), 0)

    def pb(r, tot):
        crd = coords_ref[pl.ds(r * CH, CH), :]
        trow = truth_ref[pl.ds(r, 1), :]
        own = iota2d == trow
        rn = jnp.sum(crd * crd, axis=1, keepdims=True)
        crd_aug = jnp.concatenate([crd, ones_col, rn], axis=1)
        d2c = jnp.maximum(lax.dot_general(
            cc_aug, crd_aug, (((1,), (1,)), ((), ())),
            preferred_element_type=f32), 1e-9)
        dist = d2c * lax.rsqrt(d2c)
        repm = jnp.maximum(0.0, 1.0 - dist)
        vals = jnp.where(own, d2c, repm)
        return tot + lax.dot_general(
            nonempty, vals, (((0,), (0,)), ((), ())),
            preferred_element_type=f32)

    tot_row = lax.fori_loop(0, R, pb, jnp.zeros((1, CH), f32))
    out_ref[0, 0] = jnp.sum(tot_row) * (_Q * _Q / N)


def _dense(coords, truth_rows, parts, interpret=False):
    return pl.pallas_call(
        _dense_body,
        out_shape=jax.ShapeDtypeStruct((1, 1), jnp.float32),
        in_specs=[pl.BlockSpec(memory_space=pltpu.VMEM)] * 3,
        out_specs=pl.BlockSpec(memory_space=pltpu.SMEM),
        interpret=interpret,
    )(coords, truth_rows, parts)


def kernel(x, coords, truth, row_splits):
    truth_rows = truth.reshape(R, CH).astype(jnp.int32)
    parts = _sc_segstats(coords, truth_rows)
    out = _dense(coords, truth_rows, parts)
    return out[0, 0]
